# baseline (device time: 9342 ns/iter reference)
import jax
import jax.numpy as jnp
from jax import lax
from jax.experimental import pallas as pl
from jax.experimental.pallas import tpu as pltpu

K = 8
_NEG = -3.0e38
N_CAND = 27


def _extract_cols(vals, k):
    cols = []
    for i in range(k):
        m = jnp.max(vals, axis=1, keepdims=True)
        cols.append(m)
        if i + 1 < k:
            vals = jnp.where(vals == m, _NEG, vals)
    return cols


def _fold(v):
    h = v.shape[1] // 2
    a, b = v[:, :h], v[:, h:]
    return jnp.maximum(a, b), jnp.minimum(a, b)


def _local_candidates(v):
    hi1, lo1 = _fold(v)
    hi2a, lo2a = _fold(hi1)
    hi2b, lo2b = _fold(lo1)
    pieces = []
    for arr, k in [(hi2a, 8), (lo2a, 4), (hi2b, 4), (lo2b, 2)]:
        hi, lo = _fold(arr)
        pieces.append((hi, k))
        pieces.append((lo, max(k // 2, 1)))
    cols = []
    for arr, k in pieces:
        cols.extend(_extract_cols(arr, k))
    return jnp.concatenate(cols, axis=1)


def _topk2(a, b, k):
    cols = []
    for i in range(k):
        m = jnp.maximum(
            jnp.max(a, axis=1, keepdims=True),
            jnp.max(b, axis=1, keepdims=True),
        )
        cols.append(m)
        if i + 1 < k:
            a = jnp.where(a == m, _NEG, a)
            b = jnp.where(b == m, _NEG, b)
    return jnp.concatenate(cols, axis=1)


def kernel(x):
    m, n = x.shape

    def body(x_ref, out_ref, loc_ref, rem_ref, send_sem, recv_sem):
        my_x = lax.axis_index("x")
        my_y = lax.axis_index("y")
        my_z = lax.axis_index("z")
        peer = (1 - my_x, my_y, my_z)

        barrier_sem = pltpu.get_barrier_semaphore()
        pl.semaphore_signal(
            barrier_sem, inc=1,
            device_id=peer, device_id_type=pl.DeviceIdType.MESH,
        )

        loc_ref[:, :] = _local_candidates(x_ref[:, :])

        pl.semaphore_wait(barrier_sem, 1)

        rdma = pltpu.make_async_remote_copy(
            src_ref=loc_ref,
            dst_ref=rem_ref,
            send_sem=send_sem,
            recv_sem=recv_sem,
            device_id=peer,
            device_id_type=pl.DeviceIdType.MESH,
        )
        rdma.start()
        rdma.wait_recv()

        out_ref[:, :] = _topk2(loc_ref[:, :], rem_ref[:, :], K)

        rdma.wait_send()

    return pl.pallas_call(
        body,
        out_shape=jax.ShapeDtypeStruct((m, K), jnp.float32),
        in_specs=[pl.BlockSpec(memory_space=pltpu.VMEM)],
        out_specs=pl.BlockSpec(memory_space=pltpu.VMEM),
        scratch_shapes=[
            pltpu.VMEM((m, N_CAND), jnp.float32),
            pltpu.VMEM((m, N_CAND), jnp.float32),
            pltpu.SemaphoreType.DMA,
            pltpu.SemaphoreType.DMA,
        ],
        compiler_params=pltpu.CompilerParams(collective_id=0),
    )(x)


# device time: 8650 ns/iter; 1.0800x vs baseline; 1.0800x over previous
import jax
import jax.numpy as jnp
from jax import lax
from jax.experimental import pallas as pl
from jax.experimental.pallas import tpu as pltpu

K = 8
_NEG = -3.0e38


def _topk_cols(vals, k):
    cols = []
    for i in range(k):
        m = jnp.max(vals, axis=1, keepdims=True)
        cols.append(m)
        if i + 1 < k:
            vals = jnp.where(vals == m, _NEG, vals)
    return jnp.concatenate(cols, axis=1)


def kernel(x):
    m, n = x.shape

    def body(x_ref, out_ref, loc_ref, rem_ref, send_sem, recv_sem):
        my_x = lax.axis_index("x")
        my_y = lax.axis_index("y")
        my_z = lax.axis_index("z")
        peer = (1 - my_x, my_y, my_z)

        barrier_sem = pltpu.get_barrier_semaphore()
        pl.semaphore_signal(
            barrier_sem, inc=1,
            device_id=peer, device_id_type=pl.DeviceIdType.MESH,
        )

        loc_ref[:, :] = _topk_cols(x_ref[:, :], K)

        pl.semaphore_wait(barrier_sem, 1)

        rdma = pltpu.make_async_remote_copy(
            src_ref=loc_ref,
            dst_ref=rem_ref,
            send_sem=send_sem,
            recv_sem=recv_sem,
            device_id=peer,
            device_id_type=pl.DeviceIdType.MESH,
        )
        rdma.start()
        rdma.wait_recv()

        a = loc_ref[:, :]
        b = rem_ref[:, :]
        cols = []
        for i in range(K):
            mv = jnp.maximum(
                jnp.max(a, axis=1, keepdims=True),
                jnp.max(b, axis=1, keepdims=True),
            )
            cols.append(mv)
            if i + 1 < K:
                a = jnp.where(a == mv, _NEG, a)
                b = jnp.where(b == mv, _NEG, b)
        out_ref[:, :] = jnp.concatenate(cols, axis=1)

        rdma.wait_send()

    return pl.pallas_call(
        body,
        out_shape=jax.ShapeDtypeStruct((m, K), jnp.float32),
        in_specs=[pl.BlockSpec(memory_space=pltpu.VMEM)],
        out_specs=pl.BlockSpec(memory_space=pltpu.VMEM),
        scratch_shapes=[
            pltpu.VMEM((m, K), jnp.float32),
            pltpu.VMEM((m, K), jnp.float32),
            pltpu.SemaphoreType.DMA,
            pltpu.SemaphoreType.DMA,
        ],
        compiler_params=pltpu.CompilerParams(collective_id=0),
    )(x)


# device time: 8307 ns/iter; 1.1246x vs baseline; 1.0413x over previous
import jax
import jax.numpy as jnp
from jax import lax
from jax.experimental import pallas as pl
from jax.experimental.pallas import tpu as pltpu

K = 8
_NEG = -3.0e38
N_WAVES = 2


def _topk_cols(vals, k):
    cols = []
    for i in range(k):
        m = jnp.max(vals, axis=1, keepdims=True)
        cols.append(m)
        if i + 1 < k:
            vals = jnp.where(vals == m, _NEG, vals)
    return jnp.concatenate(cols, axis=1)


def kernel(x):
    m, n = x.shape
    rows = m // N_WAVES

    def body(x_ref, out_ref, loc_ref, rem_ref, send_sems, recv_sems):
        my_x = lax.axis_index("x")
        my_y = lax.axis_index("y")
        my_z = lax.axis_index("z")
        peer = (1 - my_x, my_y, my_z)

        barrier_sem = pltpu.get_barrier_semaphore()
        pl.semaphore_signal(
            barrier_sem, inc=1,
            device_id=peer, device_id_type=pl.DeviceIdType.MESH,
        )

        rdmas = []
        for w in range(N_WAVES):
            sl = pl.ds(w * rows, rows)
            loc_ref[sl, :] = _topk_cols(x_ref[sl, :], K)
            if w == 0:
                pl.semaphore_wait(barrier_sem, 1)
            rdma = pltpu.make_async_remote_copy(
                src_ref=loc_ref.at[sl],
                dst_ref=rem_ref.at[sl],
                send_sem=send_sems.at[w],
                recv_sem=recv_sems.at[w],
                device_id=peer,
                device_id_type=pl.DeviceIdType.MESH,
            )
            rdma.start()
            rdmas.append(rdma)

        for w, rdma in enumerate(rdmas):
            sl = pl.ds(w * rows, rows)
            rdma.wait_recv()
            both = jnp.concatenate([loc_ref[sl, :], rem_ref[sl, :]], axis=1)
            out_ref[sl, :] = _topk_cols(both, K)

        for rdma in rdmas:
            rdma.wait_send()

    return pl.pallas_call(
        body,
        out_shape=jax.ShapeDtypeStruct((m, K), jnp.float32),
        in_specs=[pl.BlockSpec(memory_space=pltpu.VMEM)],
        out_specs=pl.BlockSpec(memory_space=pltpu.VMEM),
        scratch_shapes=[
            pltpu.VMEM((m, K), jnp.float32),
            pltpu.VMEM((m, K), jnp.float32),
            pltpu.SemaphoreType.DMA((N_WAVES,)),
            pltpu.SemaphoreType.DMA((N_WAVES,)),
        ],
        compiler_params=pltpu.CompilerParams(collective_id=0),
    )(x)
